# baseline (speedup 1.0000x reference)
import jax
import jax.numpy as jnp
from jax import lax
from jax.experimental import pallas as pl
from jax.experimental.pallas import tpu as pltpu

NUM_N = 10
B = 28


def _kern(img_hbm, syn_hbm, out_ref, img_ref, syn_ref, sem_i, sem_s):
    ci = pltpu.make_async_copy(img_hbm, img_ref, sem_i)
    cs = pltpu.make_async_copy(syn_hbm, syn_ref, sem_s)
    ci.start()
    cs.start()
    ci.wait()
    rows = [img_ref[pl.ds(b * B, B)] for b in range(B)]
    img2d = jnp.stack(rows, axis=0)
    basal = jnp.where(img2d > 0.5, 1.0, 0.0)
    cs.wait()
    scores = []
    for n in range(NUM_N):
        s = jnp.sum(syn_ref[n] * basal)
        scores.append(s * 16.0 + float(NUM_N - 1 - n))
    while len(scores) > 1:
        nxt = [jnp.maximum(scores[i], scores[i + 1]) for i in range(0, len(scores) - 1, 2)]
        if len(scores) % 2:
            nxt.append(scores[-1])
        scores = nxt
    best = scores[0].astype(jnp.int32)
    out_ref[0] = NUM_N - 1 - (best & 15)


def kernel(image, synapses):
    out = pl.pallas_call(
        _kern,
        out_shape=jax.ShapeDtypeStruct((1,), jnp.int32),
        in_specs=[
            pl.BlockSpec(memory_space=pl.ANY),
            pl.BlockSpec(memory_space=pl.ANY),
        ],
        out_specs=pl.BlockSpec(memory_space=pltpu.SMEM),
        scratch_shapes=[
            pltpu.VMEM((784,), jnp.float32),
            pltpu.VMEM((NUM_N, B, B), jnp.float32),
            pltpu.SemaphoreType.DMA,
            pltpu.SemaphoreType.DMA,
        ],
    )(image, synapses)
    return out[0]


# speedup vs baseline: 1.0187x; 1.0187x over previous
import jax
import jax.numpy as jnp
from jax import lax
from jax.experimental import pallas as pl
from jax.experimental.pallas import tpu as pltpu

NUM_N = 10
B = 28


def _kern(img_hbm, syn_hbm, out_ref, img_ref, syn_ref, sem_i, sem_s):
    ci = pltpu.make_async_copy(img_hbm, img_ref, sem_i)
    cs = pltpu.make_async_copy(syn_hbm, syn_ref, sem_s)
    ci.start()
    cs.start()
    ci.wait()
    rows = [img_ref[pl.ds(b * B, B)] for b in range(B)]
    img2d = jnp.stack(rows, axis=0)
    basal = jnp.where(img2d > 0.5, 1.0, 0.0)
    cs.wait()
    syn = syn_ref[...]
    t = syn * basal[None, :, :]
    s1 = jnp.sum(t, axis=1)
    firing = jnp.sum(s1, axis=1, keepdims=True)
    iota = lax.broadcasted_iota(jnp.int32, (NUM_N, 1), 0)
    score = firing * 16.0 + (NUM_N - 1 - iota).astype(jnp.float32)
    m = jnp.max(score, axis=(0, 1), keepdims=True).astype(jnp.int32)
    out_ref[...] = NUM_N - 1 - (m & 15)


def kernel(image, synapses):
    out = pl.pallas_call(
        _kern,
        out_shape=jax.ShapeDtypeStruct((1, 1), jnp.int32),
        in_specs=[
            pl.BlockSpec(memory_space=pl.ANY),
            pl.BlockSpec(memory_space=pl.ANY),
        ],
        out_specs=pl.BlockSpec(memory_space=pltpu.VMEM),
        scratch_shapes=[
            pltpu.VMEM((784,), jnp.float32),
            pltpu.VMEM((NUM_N, B, B), jnp.float32),
            pltpu.SemaphoreType.DMA,
            pltpu.SemaphoreType.DMA,
        ],
    )(image, synapses)
    return out[0, 0]


# speedup vs baseline: 1.0315x; 1.0126x over previous
import jax
import jax.numpy as jnp
from jax import lax
from jax.experimental import pallas as pl
from jax.experimental.pallas import tpu as pltpu

NUM_N = 10
B = 28


def _kern(img_hbm, syn_hbm, out_ref, img_ref, syn_ref, sem_i, sem_s):
    ci = pltpu.make_async_copy(img_hbm, img_ref, sem_i)
    cs = pltpu.make_async_copy(syn_hbm, syn_ref, sem_s)
    ci.start()
    cs.start()
    ci.wait()
    rows = [img_ref[pl.ds(b * B, B)] for b in range(B)]
    img2d = jnp.stack(rows, axis=0)
    basal16 = jnp.where(img2d > 0.5, 16.0, 0.0)
    cs.wait()
    syn = syn_ref[...]
    t = syn * basal16[None, :, :]
    s1 = jnp.sum(t, axis=1)
    firing16 = jnp.sum(s1, axis=1, keepdims=True)
    iota = lax.broadcasted_iota(jnp.int32, (NUM_N, 1), 0)
    score = firing16 + (NUM_N - 1 - iota).astype(jnp.float32)
    m = jnp.max(score, axis=(0, 1), keepdims=True).astype(jnp.int32)
    out_ref[...] = NUM_N - 1 - (m & 15)


def kernel(image, synapses):
    out = pl.pallas_call(
        _kern,
        out_shape=jax.ShapeDtypeStruct((1, 1), jnp.int32),
        in_specs=[
            pl.BlockSpec(memory_space=pl.ANY),
            pl.BlockSpec(memory_space=pl.ANY),
        ],
        out_specs=pl.BlockSpec(memory_space=pltpu.VMEM),
        scratch_shapes=[
            pltpu.VMEM((784,), jnp.float32),
            pltpu.VMEM((NUM_N, B, B), jnp.float32),
            pltpu.SemaphoreType.DMA,
            pltpu.SemaphoreType.DMA,
        ],
    )(image, synapses)
    return out[0, 0]


# speedup vs baseline: 1.0328x; 1.0013x over previous
import jax
import jax.numpy as jnp
from jax import lax
from jax.experimental import pallas as pl
from jax.experimental.pallas import tpu as pltpu

NUM_N = 10
B = 28


def _kern(img_hbm, syn_hbm, out_ref, img_ref, syn_ref, sem_i, sem_s):
    ci = pltpu.make_async_copy(img_hbm, img_ref, sem_i)
    cs = pltpu.make_async_copy(syn_hbm, syn_ref, sem_s)
    ci.start()
    cs.start()
    ci.wait()
    bin1d = jnp.where(img_ref[...] > 0.5, 16.0, 0.0)
    rows = [lax.slice(bin1d, (b * B,), (b * B + B,)) for b in range(B)]
    basal16 = jnp.stack(rows, axis=0)
    cs.wait()
    syn = syn_ref[...]
    t = syn * basal16[None, :, :]
    s1 = jnp.sum(t, axis=1)
    firing16 = jnp.sum(s1, axis=1, keepdims=True)
    iota = lax.broadcasted_iota(jnp.int32, (NUM_N, 1), 0)
    score = firing16 + (NUM_N - 1 - iota).astype(jnp.float32)
    m = jnp.max(score, axis=(0, 1), keepdims=True).astype(jnp.int32)
    out_ref[...] = NUM_N - 1 - (m & 15)


def kernel(image, synapses):
    out = pl.pallas_call(
        _kern,
        out_shape=jax.ShapeDtypeStruct((1, 1), jnp.int32),
        in_specs=[
            pl.BlockSpec(memory_space=pl.ANY),
            pl.BlockSpec(memory_space=pl.ANY),
        ],
        out_specs=pl.BlockSpec(memory_space=pltpu.VMEM),
        scratch_shapes=[
            pltpu.VMEM((784,), jnp.float32),
            pltpu.VMEM((NUM_N, B, B), jnp.float32),
            pltpu.SemaphoreType.DMA,
            pltpu.SemaphoreType.DMA,
        ],
    )(image, synapses)
    return out[0, 0]


# speedup vs baseline: 1.0357x; 1.0027x over previous
import jax
import jax.numpy as jnp
from jax import lax
from jax.experimental import pallas as pl
from jax.experimental.pallas import tpu as pltpu

NUM_N = 10
B = 28


def _kern(img_hbm, syn_hbm, out_ref, img_ref, syn_ref, sem_i, sem_s):
    ci = pltpu.make_async_copy(img_hbm, img_ref, sem_i)
    cs = pltpu.make_async_copy(syn_hbm, syn_ref, sem_s)
    ci.start()
    cs.start()
    ci.wait()
    rows = [img_ref[pl.ds(b * B, B)] for b in range(B)]
    img2d = jnp.stack(rows, axis=0)
    basal16 = jnp.where(img2d > 0.5, 16.0, 0.0)
    cs.wait()
    syn = syn_ref[...]
    t = syn * basal16[None, :, :]
    s1 = jnp.sum(t, axis=1)
    firing16 = jnp.sum(s1, axis=1, keepdims=True)
    iota = lax.broadcasted_iota(jnp.int32, (NUM_N, 1), 0)
    score = firing16 + (NUM_N - 1 - iota).astype(jnp.float32)
    m = jnp.max(score, axis=(0, 1), keepdims=True).astype(jnp.int32)
    out_ref[...] = NUM_N - 1 - (m & 15)


def kernel(image, synapses):
    out = pl.pallas_call(
        _kern,
        out_shape=jax.ShapeDtypeStruct((1, 1), jnp.int32),
        in_specs=[
            pl.BlockSpec(memory_space=pl.ANY),
            pl.BlockSpec(memory_space=pl.ANY),
        ],
        out_specs=pl.BlockSpec(memory_space=pltpu.VMEM),
        scratch_shapes=[
            pltpu.VMEM((784,), jnp.float32),
            pltpu.VMEM((NUM_N, B, B), jnp.float32),
            pltpu.SemaphoreType.DMA,
            pltpu.SemaphoreType.DMA,
        ],
    )(image, synapses)
    return out[0, 0]
